# baseline (speedup 1.0000x reference)
import jax
import jax.numpy as jnp
from jax.experimental import pallas as pl
from jax.experimental.pallas import tpu as pltpu

DM = 1024
DS = 16384
NT = 128
KK = 64

BN = 1024
NB = DS // BN


def _encode_body(x_ref, bdec_ref, w_ref, benc_ref, out_ref):
    xc = x_ref[...] - bdec_ref[...]
    out_ref[...] = (
        jnp.dot(xc, w_ref[...], preferred_element_type=jnp.float32)
        + benc_ref[...]
    )


def _topk_body(pre_ref, acts_ref):
    pre = pre_ref[...]
    bits = pltpu.bitcast(pre, jnp.int32)
    key = jnp.where(bits < 0, bits ^ 0x7FFFFFFF, bits)
    lo0 = jnp.min(key, axis=1, keepdims=True) - 1
    hi0 = jnp.max(key, axis=1, keepdims=True) + 1

    def body(_, carry):
        lo, hi = carry
        mid = (lo >> 1) + (hi >> 1) + (lo & hi & 1)
        cnt = jnp.sum((key >= mid).astype(jnp.int32), axis=1, keepdims=True)
        ok = cnt >= KK
        lo = jnp.where(ok, mid, lo)
        hi = jnp.where(ok, hi, mid)
        return lo, hi

    lo, _ = jax.lax.fori_loop(0, 32, body, (lo0, hi0))
    acts_ref[...] = jnp.where(key >= lo, jnp.maximum(pre, 0.0), 0.0)


def _decode_body(acts_ref, w_ref, bdec_ref, out_ref, acc_ref):
    j = pl.program_id(0)

    @pl.when(j == 0)
    def _():
        acc_ref[...] = jnp.zeros_like(acc_ref)

    acc_ref[...] += jnp.dot(
        acts_ref[...].astype(jnp.bfloat16),
        w_ref[...].astype(jnp.bfloat16),
        preferred_element_type=jnp.float32,
    )

    @pl.when(j == NB - 1)
    def _():
        out_ref[...] = acc_ref[...] + bdec_ref[...]


def kernel(x, W_enc, b_enc, W_dec, b_dec):
    b_enc2 = b_enc.reshape(1, DS)
    b_dec2 = b_dec.reshape(1, DM)

    pre = pl.pallas_call(
        _encode_body,
        grid=(NB,),
        in_specs=[
            pl.BlockSpec((NT, DM), lambda j: (0, 0)),
            pl.BlockSpec((1, DM), lambda j: (0, 0)),
            pl.BlockSpec((DM, BN), lambda j: (0, j)),
            pl.BlockSpec((1, BN), lambda j: (0, j)),
        ],
        out_specs=pl.BlockSpec((NT, BN), lambda j: (0, j)),
        out_shape=jax.ShapeDtypeStruct((NT, DS), jnp.float32),
    )(x, b_dec2, W_enc, b_enc2)

    acts = pl.pallas_call(
        _topk_body,
        out_shape=jax.ShapeDtypeStruct((NT, DS), jnp.float32),
    )(pre)

    recon = pl.pallas_call(
        _decode_body,
        grid=(NB,),
        in_specs=[
            pl.BlockSpec((NT, BN), lambda j: (0, j)),
            pl.BlockSpec((BN, DM), lambda j: (j, 0)),
            pl.BlockSpec((1, DM), lambda j: (0, 0)),
        ],
        out_specs=pl.BlockSpec((NT, DM), lambda j: (0, 0)),
        out_shape=jax.ShapeDtypeStruct((NT, DM), jnp.float32),
        scratch_shapes=[pltpu.VMEM((NT, DM), jnp.float32)],
    )(acts, W_dec, b_dec2)

    return (recon, acts)


# speedup vs baseline: 1.1475x; 1.1475x over previous
import jax
import jax.numpy as jnp
from jax.experimental import pallas as pl
from jax.experimental.pallas import tpu as pltpu

DM = 1024
DS = 16384
NT = 128
KK = 64

BN = 1024
NB = DS // BN

KEY_TINY = 1
KEY_INF = 0x7F800000
KEY_TS = [(127 - 2) << 23, 127 << 23, (127 + 2) << 23, (127 + 4) << 23]
NPLANES = 1 + len(KEY_TS)


def _enc_body(x_ref, bdec_ref, w_ref, benc_ref, acts_ref, key_ref, cnt_ref):
    j = pl.program_id(0)
    xc = x_ref[...] - bdec_ref[...]
    pre = (
        jnp.dot(xc, w_ref[...], preferred_element_type=jnp.float32)
        + benc_ref[...]
    )
    bits = pltpu.bitcast(pre, jnp.int32)
    key = jnp.where(bits < 0, bits ^ 0x7FFFFFFF, bits)
    key_ref[:, pl.ds(j * BN, BN)] = key

    @pl.when(j == 0)
    def _():
        cnt_ref[...] = jnp.zeros_like(cnt_ref)

    for idx, kt in enumerate([KEY_TINY] + KEY_TS):
        cnt_ref[idx] += (key >= kt).astype(jnp.int32)

    @pl.when(j == NB - 1)
    def _():
        lo = jnp.full((NT, 1), KEY_TINY, jnp.int32)
        hi = jnp.full((NT, 1), KEY_INF, jnp.int32)
        cpos = jnp.sum(cnt_ref[0], axis=1, keepdims=True)
        hi = jnp.where(cpos < KK, lo, hi)
        for idx, kt in enumerate(KEY_TS):
            c = jnp.sum(cnt_ref[idx + 1], axis=1, keepdims=True)
            lo = jnp.where(c >= KK, kt, lo)
            hi = jnp.where(c < KK, jnp.minimum(hi, kt), hi)

        keys = key_ref[...]

        def cond(carry):
            i, lo_, hi_ = carry
            return jnp.logical_and(i < 34, jnp.any(hi_ - lo_ > 1))

        def body(carry):
            i, lo_, hi_ = carry
            mid = (lo_ >> 1) + (hi_ >> 1) + (lo_ & hi_ & 1)
            cnt = jnp.sum((keys >= mid).astype(jnp.int32), axis=1,
                          keepdims=True)
            ok = cnt >= KK
            return (
                i + 1,
                jnp.where(ok, mid, lo_),
                jnp.where(ok, hi_, mid),
            )

        _, lo, _ = jax.lax.while_loop(
            cond, body, (jnp.int32(0), lo, hi)
        )
        acts_ref[...] = jnp.where(
            keys >= lo, pltpu.bitcast(keys, jnp.float32), 0.0
        )


def _decode_body(acts_ref, w_ref, bdec_ref, out_ref, acc_ref):
    j = pl.program_id(0)

    @pl.when(j == 0)
    def _():
        acc_ref[...] = jnp.zeros_like(acc_ref)

    acc_ref[...] += jnp.dot(
        acts_ref[...], w_ref[...], preferred_element_type=jnp.float32
    )

    @pl.when(j == NB - 1)
    def _():
        out_ref[...] = acc_ref[...] + bdec_ref[...]


def kernel(x, W_enc, b_enc, W_dec, b_dec):
    b_enc2 = b_enc.reshape(1, DS)
    b_dec2 = b_dec.reshape(1, DM)

    acts = pl.pallas_call(
        _enc_body,
        grid=(NB,),
        in_specs=[
            pl.BlockSpec((NT, DM), lambda j: (0, 0)),
            pl.BlockSpec((1, DM), lambda j: (0, 0)),
            pl.BlockSpec((DM, BN), lambda j: (0, j)),
            pl.BlockSpec((1, BN), lambda j: (0, j)),
        ],
        out_specs=pl.BlockSpec((NT, DS), lambda j: (0, 0)),
        out_shape=jax.ShapeDtypeStruct((NT, DS), jnp.float32),
        scratch_shapes=[
            pltpu.VMEM((NT, DS), jnp.int32),
            pltpu.VMEM((NPLANES, NT, BN), jnp.int32),
        ],
    )(x, b_dec2, W_enc, b_enc2)

    recon = pl.pallas_call(
        _decode_body,
        grid=(NB,),
        in_specs=[
            pl.BlockSpec((NT, BN), lambda j: (0, j)),
            pl.BlockSpec((BN, DM), lambda j: (j, 0)),
            pl.BlockSpec((1, DM), lambda j: (0, 0)),
        ],
        out_specs=pl.BlockSpec((NT, DM), lambda j: (0, 0)),
        out_shape=jax.ShapeDtypeStruct((NT, DM), jnp.float32),
        scratch_shapes=[pltpu.VMEM((NT, DM), jnp.float32)],
    )(acts, W_dec, b_dec2)

    return (recon, acts)


# speedup vs baseline: 1.1756x; 1.0245x over previous
import jax
import jax.numpy as jnp
from jax.experimental import pallas as pl
from jax.experimental.pallas import tpu as pltpu

DM = 1024
DS = 16384
NT = 128
KK = 64

BN = 1024
NB = DS // BN

KEY_TINY = 1
KEY_INF = 0x7F800000
KEY_TS = [(127 - 2) << 23, 127 << 23, (127 + 2) << 23, (127 + 4) << 23]
NPLANES = 1 + len(KEY_TS)


def _body(x_ref, bdec_ref, wenc_ref, benc_ref, wdec_ref,
          acts_ref, recon_ref, key_ref, cnt_ref, lo_ref, acc_ref):
    j = pl.program_id(0)

    @pl.when(j < NB)
    def _encode():
        xc = x_ref[...] - bdec_ref[...]
        pre = (
            jnp.dot(xc, wenc_ref[...], preferred_element_type=jnp.float32)
            + benc_ref[...]
        )
        bits = pltpu.bitcast(pre, jnp.int32)
        key = jnp.where(bits < 0, bits ^ 0x7FFFFFFF, bits)
        key_ref[:, pl.ds(j * BN, BN)] = key

        @pl.when(j == 0)
        def _():
            cnt_ref[...] = jnp.zeros_like(cnt_ref)

        for idx, kt in enumerate([KEY_TINY] + KEY_TS):
            cnt_ref[idx] += (key >= kt).astype(jnp.int32)

    @pl.when(j == NB - 1)
    def _select():
        lo = jnp.full((NT, 1), KEY_TINY, jnp.int32)
        hi = jnp.full((NT, 1), KEY_INF, jnp.int32)
        cpos = jnp.sum(cnt_ref[0], axis=1, keepdims=True)
        hi = jnp.where(cpos < KK, lo, hi)
        for idx, kt in enumerate(KEY_TS):
            c = jnp.sum(cnt_ref[idx + 1], axis=1, keepdims=True)
            lo = jnp.where(c >= KK, kt, lo)
            hi = jnp.where(c < KK, jnp.minimum(hi, kt), hi)

        keys = key_ref[...]

        def cond(carry):
            i, lo_, hi_ = carry
            return jnp.logical_and(i < 34, jnp.any(hi_ - lo_ > 1))

        def body(carry):
            i, lo_, hi_ = carry
            mid = (lo_ >> 1) + (hi_ >> 1) + (lo_ & hi_ & 1)
            cnt = jnp.sum((keys >= mid).astype(jnp.int32), axis=1,
                          keepdims=True)
            ok = cnt >= KK
            return (
                i + 1,
                jnp.where(ok, mid, lo_),
                jnp.where(ok, hi_, mid),
            )

        _, lo, _ = jax.lax.while_loop(
            cond, body, (jnp.int32(0), lo, hi)
        )
        lo_ref[...] = lo

    @pl.when(j >= NB)
    def _decode():
        jd = j - NB
        key = key_ref[:, pl.ds(jd * BN, BN)]
        a = jnp.where(
            key >= lo_ref[...], pltpu.bitcast(key, jnp.float32), 0.0
        )
        acts_ref[...] = a

        @pl.when(j == NB)
        def _():
            acc_ref[...] = jnp.zeros_like(acc_ref)

        acc_ref[...] += jnp.dot(
            a, wdec_ref[...], preferred_element_type=jnp.float32
        )

        @pl.when(j == 2 * NB - 1)
        def _():
            recon_ref[...] = acc_ref[...] + bdec_ref[...]


def kernel(x, W_enc, b_enc, W_dec, b_dec):
    b_enc2 = b_enc.reshape(1, DS)
    b_dec2 = b_dec.reshape(1, DM)

    acts, recon = pl.pallas_call(
        _body,
        grid=(2 * NB,),
        in_specs=[
            pl.BlockSpec((NT, DM), lambda j: (0, 0)),
            pl.BlockSpec((1, DM), lambda j: (0, 0)),
            pl.BlockSpec((DM, BN), lambda j: (0, jnp.minimum(j, NB - 1))),
            pl.BlockSpec((1, BN), lambda j: (0, jnp.minimum(j, NB - 1))),
            pl.BlockSpec(
                (BN, DM), lambda j: (jnp.maximum(j - NB, 0), 0)
            ),
        ],
        out_specs=[
            pl.BlockSpec(
                (NT, BN), lambda j: (0, jnp.maximum(j - NB, 0))
            ),
            pl.BlockSpec((NT, DM), lambda j: (0, 0)),
        ],
        out_shape=[
            jax.ShapeDtypeStruct((NT, DS), jnp.float32),
            jax.ShapeDtypeStruct((NT, DM), jnp.float32),
        ],
        scratch_shapes=[
            pltpu.VMEM((NT, DS), jnp.int32),
            pltpu.VMEM((NPLANES, NT, BN), jnp.int32),
            pltpu.VMEM((NT, 1), jnp.int32),
            pltpu.VMEM((NT, DM), jnp.float32),
        ],
    )(x, b_dec2, W_enc, b_enc2, W_dec)

    return (recon, acts)
